# Initial kernel scaffold; baseline (speedup 1.0000x reference)
import jax
import jax.numpy as jnp
from jax.experimental import pallas as pl


def kernel(tracks, W_ne, b_ne, W_ep1, b_ep1, W_ep2, b_ep2, W_ee, b_ee, W_nm, b_nm, ln_g, ln_b, W_em, b_em, W_gp1, b_gp1, W_gp2, b_gp2):
    raise NotImplementedError("write your pallas kernel here")


# speedup vs baseline: 4.2728x; 4.2728x over previous
import numpy as np
import jax
import jax.numpy as jnp
from jax.experimental import pallas as pl
from jax.experimental.pallas import tpu as pltpu

_N = 128
_D = 1024
_ED = 256
_L = 3
_TI = 8
_NT = _N // _TI


def _gelu(x):
    return 0.5 * x * (1.0 + jax.lax.erf(x * np.float32(1.0 / np.sqrt(2.0))))


def _prep_body(tracks_ref, W_ne_ref, b_ne_ref, W_ep1_ref, W_ee_ref, b_ee_ref,
               W_em_ref, b_em_ref,
               nodes_ref, a_ref, b_ref, wc_ref, bc_ref):
    nodes = jnp.dot(tracks_ref[...], W_ne_ref[...],
                    preferred_element_type=jnp.float32) + b_ne_ref[...]
    nodes_ref[...] = nodes
    a_ref[...] = jnp.dot(nodes, W_ep1_ref[:_D, :],
                         preferred_element_type=jnp.float32)
    b_ref[...] = jnp.dot(nodes, W_ep1_ref[_D:, :],
                         preferred_element_type=jnp.float32)
    for l in range(_L):
        wc_ref[l] = jnp.dot(W_ee_ref[...], W_em_ref[l],
                            preferred_element_type=jnp.float32)
        bc_ref[pl.ds(l, 1), :] = (jnp.dot(b_ee_ref[...], W_em_ref[l],
                                          preferred_element_type=jnp.float32)
                                  + b_em_ref[pl.ds(l, 1), :])


def _edges_body(a_ref, b_ref, b_ep1_ref, W_ep2_ref, b_ep2_ref,
                W_ee_ref, b_ee_ref,
                ef_ref, full_ref):
    a = a_ref[...]
    b = b_ref[...]
    pre = a[:, None, :] + b[None, :, :] + b_ep1_ref[...][None, :, :]
    pre = pre.reshape(_TI * _N, _ED)
    ef = jnp.dot(_gelu(pre), W_ep2_ref[...],
                 preferred_element_type=jnp.float32) + b_ep2_ref[...]
    ef_ref[...] = ef
    full_ref[...] = jnp.dot(ef, W_ee_ref[...],
                            preferred_element_type=jnp.float32) + b_ee_ref[...]


def _layer_body(ef_ref, x_ref, wc_ref, bc_ref, wnm_a_ref, wnm_b_ref,
                b_nm_ref, ln_g_ref, ln_b_ref,
                x_new_ref, acc_ref):
    t = pl.program_id(0)
    g = _gelu(jnp.dot(ef_ref[...], wc_ref[...],
                      preferred_element_type=jnp.float32) + bc_ref[...])
    g = g.reshape(_TI, _N, _D)
    xt = x_ref[pl.ds(t * _TI, _TI), :]
    jcol = jax.lax.broadcasted_iota(jnp.int32, (_TI, _N), 1)
    isrc = t * _TI + jax.lax.broadcasted_iota(jnp.int32, (_TI, _N), 0)
    keep = (jcol != isrc).astype(jnp.float32)[:, :, None]
    contrib = jnp.sum(g * keep * xt[:, None, :], axis=0)

    @pl.when(t == 0)
    def _():
        acc_ref[...] = contrib

    @pl.when(t > 0)
    def _():
        acc_ref[...] = acc_ref[...] + contrib

    @pl.when(t == _NT - 1)
    def _():
        agg = acc_ref[...] * (1.0 / 127.0)
        x = x_ref[...]
        h = (jnp.dot(x, wnm_a_ref[...], preferred_element_type=jnp.float32)
             + jnp.dot(agg, wnm_b_ref[...], preferred_element_type=jnp.float32)
             + b_nm_ref[...])
        mu = jnp.mean(h, axis=1, keepdims=True)
        var = jnp.mean((h - mu) * (h - mu), axis=1, keepdims=True)
        hn = (h - mu) / jnp.sqrt(var + 1e-5) * ln_g_ref[...] + ln_b_ref[...]
        x_new_ref[...] = _gelu(hn)


def _pool_body(x_ref, W_gp1_ref, b_gp1_ref, W_gp2_ref, b_gp2_ref, out_ref):
    gmean = jnp.mean(x_ref[...], axis=0, keepdims=True)
    h = _gelu(jnp.dot(gmean, W_gp1_ref[...],
                      preferred_element_type=jnp.float32) + b_gp1_ref[...])
    out_ref[...] = jnp.dot(h, W_gp2_ref[...],
                           preferred_element_type=jnp.float32) + b_gp2_ref[...]


def kernel(tracks, W_ne, b_ne, W_ep1, b_ep1, W_ep2, b_ep2, W_ee, b_ee,
           W_nm, b_nm, ln_g, ln_b, W_em, b_em, W_gp1, b_gp1, W_gp2, b_gp2):
    f32 = jnp.float32
    b_ne2 = b_ne.reshape(1, _D)
    b_ep1_2 = b_ep1.reshape(1, _ED)
    b_ep2_2 = b_ep2.reshape(1, _ED)
    b_ee2 = b_ee.reshape(1, _D)
    b_gp1_2 = b_gp1.reshape(1, _D // 2)
    b_gp2_2 = b_gp2.reshape(1, _D)

    nodes, A, B, Wc, bc = pl.pallas_call(
        _prep_body,
        out_shape=(
            jax.ShapeDtypeStruct((_N, _D), f32),
            jax.ShapeDtypeStruct((_N, _ED), f32),
            jax.ShapeDtypeStruct((_N, _ED), f32),
            jax.ShapeDtypeStruct((_L, _ED, _D), f32),
            jax.ShapeDtypeStruct((_L, _D), f32),
        ),
    )(tracks, W_ne, b_ne2, W_ep1, W_ee, b_ee2, W_em, b_em)

    ef, full = pl.pallas_call(
        _edges_body,
        grid=(_NT,),
        in_specs=[
            pl.BlockSpec((_TI, _ED), lambda t: (t, 0)),
            pl.BlockSpec((_N, _ED), lambda t: (0, 0)),
            pl.BlockSpec((1, _ED), lambda t: (0, 0)),
            pl.BlockSpec((_ED, _ED), lambda t: (0, 0)),
            pl.BlockSpec((1, _ED), lambda t: (0, 0)),
            pl.BlockSpec((_ED, _D), lambda t: (0, 0)),
            pl.BlockSpec((1, _D), lambda t: (0, 0)),
        ],
        out_specs=(
            pl.BlockSpec((_TI * _N, _ED), lambda t: (t, 0)),
            pl.BlockSpec((_TI * _N, _D), lambda t: (t, 0)),
        ),
        out_shape=(
            jax.ShapeDtypeStruct((_N * _N, _ED), f32),
            jax.ShapeDtypeStruct((_N * _N, _D), f32),
        ),
    )(A, B, b_ep1_2, W_ep2, b_ep2_2, W_ee, b_ee2)

    edge_features = (
        full[1:, :].reshape(_N - 1, _N + 1, _D)[:, :_N, :].reshape(_N * (_N - 1), _D)
    )

    x = nodes
    for l in range(_L):
        x = pl.pallas_call(
            _layer_body,
            grid=(_NT,),
            in_specs=[
                pl.BlockSpec((_TI * _N, _ED), lambda t: (t, 0)),
                pl.BlockSpec((_N, _D), lambda t: (0, 0)),
                pl.BlockSpec((_ED, _D), lambda t: (0, 0)),
                pl.BlockSpec((1, _D), lambda t: (0, 0)),
                pl.BlockSpec((_D, _D), lambda t: (0, 0)),
                pl.BlockSpec((_D, _D), lambda t: (0, 0)),
                pl.BlockSpec((1, _D), lambda t: (0, 0)),
                pl.BlockSpec((1, _D), lambda t: (0, 0)),
                pl.BlockSpec((1, _D), lambda t: (0, 0)),
            ],
            out_specs=pl.BlockSpec((_N, _D), lambda t: (0, 0)),
            out_shape=jax.ShapeDtypeStruct((_N, _D), f32),
            scratch_shapes=[pltpu.VMEM((_N, _D), f32)],
        )(ef, x, Wc[l], bc[l].reshape(1, _D), W_nm[l, :_D, :], W_nm[l, _D:, :],
          b_nm[l].reshape(1, _D), ln_g[l].reshape(1, _D), ln_b[l].reshape(1, _D))

    graph_embedding = pl.pallas_call(
        _pool_body,
        out_shape=jax.ShapeDtypeStruct((1, _D), f32),
    )(x, W_gp1, b_gp1_2, W_gp2, b_gp2_2).reshape(_D)

    ii = np.repeat(np.arange(_N), _N)
    jj = np.tile(np.arange(_N), _N)
    m = ii != jj
    edge_index = jnp.stack([jnp.asarray(ii[m], dtype=jnp.int32),
                            jnp.asarray(jj[m], dtype=jnp.int32)])

    return x, edge_features, edge_index, graph_embedding


# speedup vs baseline: 7.8152x; 1.8291x over previous
import numpy as np
import jax
import jax.numpy as jnp
from jax.experimental import pallas as pl
from jax.experimental.pallas import tpu as pltpu

_N = 128
_D = 1024
_ED = 256
_L = 3
_E = _N * (_N - 1)
_TI = 8
_NT = _N // _TI
_TE = _TI * (_N - 1)
_NE = _E // _TE


def _gelu(x):
    return 0.5 * x * (1.0 + jax.lax.erf(x * np.float32(1.0 / np.sqrt(2.0))))


def _bf(x):
    return x.astype(jnp.bfloat16)


def _prep_body(tracks_ref, W_ne_ref, b_ne_ref, W_ep1_ref, b_ep1_ref,
               W_ep2_ref, b_ep2_ref, W_ee_ref, b_ee_ref, W_em_ref, b_em_ref,
               nodes_ref, a_ref, b_ref, wc_ref, bc_ref, gd_ref):
    nodes = jnp.dot(tracks_ref[...], W_ne_ref[...],
                    preferred_element_type=jnp.float32) + b_ne_ref[...]
    nodes_ref[...] = nodes
    a = jnp.dot(nodes, W_ep1_ref[:_D, :], preferred_element_type=jnp.float32)
    b = jnp.dot(nodes, W_ep1_ref[_D:, :], preferred_element_type=jnp.float32)
    a_ref[...] = a
    b_ref[...] = b
    efd = jnp.dot(_bf(_gelu(a + b + b_ep1_ref[...])), _bf(W_ep2_ref[...]),
                  preferred_element_type=jnp.float32) + b_ep2_ref[...]
    for l in range(_L):
        wc = jnp.dot(W_ee_ref[...], W_em_ref[l],
                     preferred_element_type=jnp.float32)
        bc = (jnp.dot(b_ee_ref[...], W_em_ref[l],
                      preferred_element_type=jnp.float32)
              + b_em_ref[pl.ds(l, 1), :])
        wc_ref[l] = _bf(wc)
        bc_ref[pl.ds(l, 1), :] = bc
        gd_ref[l] = _gelu(jnp.dot(_bf(efd), _bf(wc),
                                  preferred_element_type=jnp.float32) + bc)


def _edges_body(iidx_ref, jidx_ref, a_ref, b_ref, b_ep1_ref,
                W_ep2_ref, b_ep2_ref, W_ee_ref, b_ee_ref, out_ref):
    col = jax.lax.broadcasted_iota(jnp.int32, (_TE, _N), 1)
    pi = (col == iidx_ref[0, 0, :][:, None]).astype(jnp.bfloat16)
    pj = (col == jidx_ref[0, 0, :][:, None]).astype(jnp.bfloat16)
    ai = jnp.dot(pi, a_ref[...], preferred_element_type=jnp.float32)
    bj = jnp.dot(pj, b_ref[...], preferred_element_type=jnp.float32)
    pre = ai + bj + b_ep1_ref[...]
    ef = jnp.dot(_bf(_gelu(pre)), W_ep2_ref[...],
                 preferred_element_type=jnp.float32) + b_ep2_ref[...]
    out_ref[...] = jnp.dot(_bf(ef), W_ee_ref[...],
                           preferred_element_type=jnp.float32) + b_ee_ref[...]


def _layer_body(a_ref, b_ref, b_ep1_ref, W_ep2_ref, b_ep2_ref,
                x_ref, wc_ref, bc_ref, gd_ref, wnm_a_ref, wnm_b_ref,
                b_nm_ref, ln_g_ref, ln_b_ref,
                x_new_ref, acc_ref):
    t = pl.program_id(0)
    a = a_ref[...]
    b = b_ref[...]
    pre = a[:, None, :] + b[None, :, :] + b_ep1_ref[...][None, :, :]
    ef = jnp.dot(_bf(_gelu(pre.reshape(_TI * _N, _ED))), W_ep2_ref[...],
                 preferred_element_type=jnp.float32) + b_ep2_ref[...]
    g = _gelu(jnp.dot(_bf(ef), wc_ref[...],
                      preferred_element_type=jnp.float32) + bc_ref[...])
    g = g.reshape(_TI, _N, _D)
    xt = x_ref[pl.ds(t * _TI, _TI), :]
    contrib = jnp.sum(g * xt[:, None, :], axis=0)

    @pl.when(t == 0)
    def _():
        acc_ref[...] = contrib - x_ref[...] * gd_ref[...]

    @pl.when(t > 0)
    def _():
        acc_ref[...] = acc_ref[...] + contrib

    @pl.when(t == _NT - 1)
    def _():
        agg = acc_ref[...] * (1.0 / 127.0)
        x = x_ref[...]
        h = (jnp.dot(x, wnm_a_ref[...], preferred_element_type=jnp.float32)
             + jnp.dot(agg, wnm_b_ref[...], preferred_element_type=jnp.float32)
             + b_nm_ref[...])
        mu = jnp.mean(h, axis=1, keepdims=True)
        var = jnp.mean((h - mu) * (h - mu), axis=1, keepdims=True)
        hn = (h - mu) / jnp.sqrt(var + 1e-5) * ln_g_ref[...] + ln_b_ref[...]
        x_new_ref[...] = _gelu(hn)


def _pool_body(x_ref, W_gp1_ref, b_gp1_ref, W_gp2_ref, b_gp2_ref, out_ref):
    gmean = jnp.mean(x_ref[...], axis=0, keepdims=True)
    h = _gelu(jnp.dot(gmean, W_gp1_ref[...],
                      preferred_element_type=jnp.float32) + b_gp1_ref[...])
    out_ref[...] = jnp.dot(h, W_gp2_ref[...],
                           preferred_element_type=jnp.float32) + b_gp2_ref[...]


_ii = np.repeat(np.arange(_N), _N)
_jj = np.tile(np.arange(_N), _N)
_mask = _ii != _jj
_SRC = np.ascontiguousarray(_ii[_mask]).astype(np.int32)
_TGT = np.ascontiguousarray(_jj[_mask]).astype(np.int32)
_IIDX = _SRC.reshape(_NE, 1, _TE)
_JIDX = _TGT.reshape(_NE, 1, _TE)


def kernel(tracks, W_ne, b_ne, W_ep1, b_ep1, W_ep2, b_ep2, W_ee, b_ee,
           W_nm, b_nm, ln_g, ln_b, W_em, b_em, W_gp1, b_gp1, W_gp2, b_gp2):
    f32 = jnp.float32
    bf16 = jnp.bfloat16
    b_ne2 = b_ne.reshape(1, _D)
    b_ep1_2 = b_ep1.reshape(1, _ED)
    b_ep2_2 = b_ep2.reshape(1, _ED)
    b_ee2 = b_ee.reshape(1, _D)
    W_ep2_bf = W_ep2.astype(bf16)
    W_ee_bf = W_ee.astype(bf16)

    nodes, A, B, Wc, bc, gd = pl.pallas_call(
        _prep_body,
        out_shape=(
            jax.ShapeDtypeStruct((_N, _D), f32),
            jax.ShapeDtypeStruct((_N, _ED), f32),
            jax.ShapeDtypeStruct((_N, _ED), f32),
            jax.ShapeDtypeStruct((_L, _ED, _D), bf16),
            jax.ShapeDtypeStruct((_L, _D), f32),
            jax.ShapeDtypeStruct((_L, _N, _D), f32),
        ),
    )(tracks, W_ne, b_ne2, W_ep1, b_ep1_2, W_ep2, b_ep2_2, W_ee, b_ee2,
      W_em, b_em)

    edge_features = pl.pallas_call(
        _edges_body,
        grid=(_NE,),
        in_specs=[
            pl.BlockSpec((1, 1, _TE), lambda t: (t, 0, 0)),
            pl.BlockSpec((1, 1, _TE), lambda t: (t, 0, 0)),
            pl.BlockSpec((_N, _ED), lambda t: (0, 0)),
            pl.BlockSpec((_N, _ED), lambda t: (0, 0)),
            pl.BlockSpec((1, _ED), lambda t: (0, 0)),
            pl.BlockSpec((_ED, _ED), lambda t: (0, 0)),
            pl.BlockSpec((1, _ED), lambda t: (0, 0)),
            pl.BlockSpec((_ED, _D), lambda t: (0, 0)),
            pl.BlockSpec((1, _D), lambda t: (0, 0)),
        ],
        out_specs=pl.BlockSpec((_TE, _D), lambda t: (t, 0)),
        out_shape=jax.ShapeDtypeStruct((_E, _D), f32),
    )(jnp.asarray(_IIDX), jnp.asarray(_JIDX), A.astype(bf16), B.astype(bf16),
      b_ep1_2, W_ep2_bf, b_ep2_2, W_ee_bf, b_ee2)

    x = nodes
    for l in range(_L):
        x = pl.pallas_call(
            _layer_body,
            grid=(_NT,),
            in_specs=[
                pl.BlockSpec((_TI, _ED), lambda t: (t, 0)),
                pl.BlockSpec((_N, _ED), lambda t: (0, 0)),
                pl.BlockSpec((1, _ED), lambda t: (0, 0)),
                pl.BlockSpec((_ED, _ED), lambda t: (0, 0)),
                pl.BlockSpec((1, _ED), lambda t: (0, 0)),
                pl.BlockSpec((_N, _D), lambda t: (0, 0)),
                pl.BlockSpec((_ED, _D), lambda t: (0, 0)),
                pl.BlockSpec((1, _D), lambda t: (0, 0)),
                pl.BlockSpec((_N, _D), lambda t: (0, 0)),
                pl.BlockSpec((_D, _D), lambda t: (0, 0)),
                pl.BlockSpec((_D, _D), lambda t: (0, 0)),
                pl.BlockSpec((1, _D), lambda t: (0, 0)),
                pl.BlockSpec((1, _D), lambda t: (0, 0)),
                pl.BlockSpec((1, _D), lambda t: (0, 0)),
            ],
            out_specs=pl.BlockSpec((_N, _D), lambda t: (0, 0)),
            out_shape=jax.ShapeDtypeStruct((_N, _D), f32),
            scratch_shapes=[pltpu.VMEM((_N, _D), f32)],
        )(A, B, b_ep1_2, W_ep2_bf, b_ep2_2, x, Wc[l], bc[l].reshape(1, _D),
          gd[l], W_nm[l, :_D, :], W_nm[l, _D:, :],
          b_nm[l].reshape(1, _D), ln_g[l].reshape(1, _D), ln_b[l].reshape(1, _D))

    graph_embedding = pl.pallas_call(
        _pool_body,
        out_shape=jax.ShapeDtypeStruct((1, _D), f32),
    )(x, W_gp1, b_gp1.reshape(1, _D // 2), W_gp2, b_gp2.reshape(1, _D)
      ).reshape(_D)

    edge_index = jnp.stack([jnp.asarray(_SRC), jnp.asarray(_TGT)])

    return x, edge_features, edge_index, graph_embedding


# speedup vs baseline: 9.3813x; 1.2004x over previous
import numpy as np
import jax
import jax.numpy as jnp
from jax.experimental import pallas as pl
from jax.experimental.pallas import tpu as pltpu

_N = 128
_D = 1024
_ED = 256
_L = 3
_E = _N * (_N - 1)
_TI = 8
_TE = _TI * (_N - 1)
_NE = _E // _TE
_TL = 8
_NT = _N // _TL


def _gelu(x):
    return x * (0.5 * jax.lax.erf(x * np.float32(1.0 / np.sqrt(2.0))) + 0.5)


def _bf(x):
    return x.astype(jnp.bfloat16)


def _dot(a, b):
    return jnp.dot(a, b, preferred_element_type=jnp.float32)


def _prep_body(tracks_ref, W_ne_ref, b_ne_ref, W_ep1_ref, b_ep1_ref,
               W_ep2_ref, b_ep2_ref, W_ee_ref, b_ee_ref, W_em_ref, b_em_ref,
               nodes_ref, a_ref, b_ref, wc_ref, bc_ref, gd_ref, bee_ref):
    nodes = _dot(tracks_ref[...], W_ne_ref[...]) + b_ne_ref[...]
    nodes_ref[...] = nodes
    a = _dot(nodes, W_ep1_ref[:_D, :]) + b_ep1_ref[...]
    b = _dot(nodes, W_ep1_ref[_D:, :])
    a_ref[...] = a
    b_ref[...] = b
    bee = _dot(b_ep2_ref[...], W_ee_ref[...]) + b_ee_ref[...]
    bee_ref[...] = bee
    efd = _dot(_bf(_gelu(a + b)), _bf(W_ep2_ref[...]))
    for l in range(_L):
        wc = _dot(_bf(W_ee_ref[...]), _bf(W_em_ref[l]))
        bc = _dot(bee, W_em_ref[l]) + b_em_ref[pl.ds(l, 1), :]
        wc_ref[l] = _bf(wc)
        bc_ref[pl.ds(l, 1), :] = bc
        gd_ref[l] = _gelu(_dot(_bf(efd), _bf(wc)) + bc)


def _edges_body(iidx_ref, jidx_ref, a_ref, b_ref,
                W_ep2_ref, W_ee_ref, bee_ref, out_ref):
    col = jax.lax.broadcasted_iota(jnp.int32, (_TE, _N), 1)
    pi = (col == iidx_ref[0, 0, :][:, None]).astype(jnp.bfloat16)
    pj = (col == jidx_ref[0, 0, :][:, None]).astype(jnp.bfloat16)
    pre = _dot(pi, a_ref[...]) + _dot(pj, b_ref[...])
    ef0 = _dot(_bf(_gelu(pre)), W_ep2_ref[...])
    out_ref[...] = _dot(_bf(ef0), W_ee_ref[...]) + bee_ref[...]


def _layers_body(a_ref, b_ref, W_ep2_ref, nodes_ref,
                 wc_ref, bc_ref, gd_ref, wnm_ref, b_nm_ref, ln_g_ref,
                 ln_b_ref, W_gp1_ref, b_gp1_ref, W_gp2_ref, b_gp2_ref,
                 x_out_ref, ge_out_ref, x_ref, acc_ref):
    l = pl.program_id(0)
    t = pl.program_id(1)

    @pl.when((l == 0) & (t == 0))
    def _():
        x_ref[...] = nodes_ref[...]

    a = a_ref[...]
    b = b_ref[...]
    pre = a[:, None, :] + b[None, :, :]
    ef0 = _dot(_bf(_gelu(pre.reshape(_TL * _N, _ED))), W_ep2_ref[...])
    g = _gelu(_dot(_bf(ef0), wc_ref[0]) + bc_ref[0])
    g = g.reshape(_TL, _N, _D)
    xt = x_ref[pl.ds(t * _TL, _TL), :]
    contrib = jnp.sum(g * xt[:, None, :], axis=0)

    @pl.when(t == 0)
    def _():
        acc_ref[...] = contrib - x_ref[...] * gd_ref[0]

    @pl.when(t > 0)
    def _():
        acc_ref[...] = acc_ref[...] + contrib

    @pl.when(t == _NT - 1)
    def _():
        agg = acc_ref[...] * (1.0 / 127.0)
        x = x_ref[...]
        wnm = wnm_ref[0]
        h = (_dot(_bf(x), _bf(wnm[:_D, :]))
             + _dot(_bf(agg), _bf(wnm[_D:, :]))
             + b_nm_ref[0])
        mu = jnp.mean(h, axis=1, keepdims=True)
        var = jnp.mean((h - mu) * (h - mu), axis=1, keepdims=True)
        hn = (h - mu) / jnp.sqrt(var + 1e-5) * ln_g_ref[0] + ln_b_ref[0]
        xn = _gelu(hn)
        x_ref[...] = xn

        @pl.when(l == _L - 1)
        def _():
            x_out_ref[...] = xn
            gmean = jnp.mean(xn, axis=0, keepdims=True)
            hp = _gelu(_dot(gmean, W_gp1_ref[...]) + b_gp1_ref[...])
            ge_out_ref[...] = _dot(hp, W_gp2_ref[...]) + b_gp2_ref[...]


_ii = np.repeat(np.arange(_N), _N)
_jj = np.tile(np.arange(_N), _N)
_msk = _ii != _jj
_SRC = np.ascontiguousarray(_ii[_msk]).astype(np.int32)
_TGT = np.ascontiguousarray(_jj[_msk]).astype(np.int32)
_IIDX = _SRC.reshape(_NE, 1, _TE)
_JIDX = _TGT.reshape(_NE, 1, _TE)


def kernel(tracks, W_ne, b_ne, W_ep1, b_ep1, W_ep2, b_ep2, W_ee, b_ee,
           W_nm, b_nm, ln_g, ln_b, W_em, b_em, W_gp1, b_gp1, W_gp2, b_gp2):
    f32 = jnp.float32
    bf16 = jnp.bfloat16

    nodes, A, B, Wc, bc, gd, bee = pl.pallas_call(
        _prep_body,
        out_shape=(
            jax.ShapeDtypeStruct((_N, _D), f32),
            jax.ShapeDtypeStruct((_N, _ED), f32),
            jax.ShapeDtypeStruct((_N, _ED), f32),
            jax.ShapeDtypeStruct((_L, _ED, _D), bf16),
            jax.ShapeDtypeStruct((_L, _D), f32),
            jax.ShapeDtypeStruct((_L, _N, _D), f32),
            jax.ShapeDtypeStruct((1, _D), f32),
        ),
    )(tracks, W_ne, b_ne.reshape(1, _D), W_ep1, b_ep1.reshape(1, _ED),
      W_ep2, b_ep2.reshape(1, _ED), W_ee, b_ee.reshape(1, _D), W_em, b_em)

    W_ep2_bf = W_ep2.astype(bf16)

    edge_features = pl.pallas_call(
        _edges_body,
        grid=(_NE,),
        in_specs=[
            pl.BlockSpec((1, 1, _TE), lambda t: (t, 0, 0)),
            pl.BlockSpec((1, 1, _TE), lambda t: (t, 0, 0)),
            pl.BlockSpec((_N, _ED), lambda t: (0, 0)),
            pl.BlockSpec((_N, _ED), lambda t: (0, 0)),
            pl.BlockSpec((_ED, _ED), lambda t: (0, 0)),
            pl.BlockSpec((_ED, _D), lambda t: (0, 0)),
            pl.BlockSpec((1, _D), lambda t: (0, 0)),
        ],
        out_specs=pl.BlockSpec((_TE, _D), lambda t: (t, 0)),
        out_shape=jax.ShapeDtypeStruct((_E, _D), f32),
    )(jnp.asarray(_IIDX), jnp.asarray(_JIDX), A.astype(bf16), B.astype(bf16),
      W_ep2_bf, W_ee.astype(bf16), bee)

    x, graph_embedding = pl.pallas_call(
        _layers_body,
        grid=(_L, _NT),
        in_specs=[
            pl.BlockSpec((_TL, _ED), lambda l, t: (t, 0)),
            pl.BlockSpec((_N, _ED), lambda l, t: (0, 0)),
            pl.BlockSpec((_ED, _ED), lambda l, t: (0, 0)),
            pl.BlockSpec((_N, _D), lambda l, t: (0, 0)),
            pl.BlockSpec((1, _ED, _D), lambda l, t: (l, 0, 0)),
            pl.BlockSpec((1, 1, _D), lambda l, t: (l, 0, 0)),
            pl.BlockSpec((1, _N, _D), lambda l, t: (l, 0, 0)),
            pl.BlockSpec((1, 2 * _D, _D), lambda l, t: (l, 0, 0)),
            pl.BlockSpec((1, 1, _D), lambda l, t: (l, 0, 0)),
            pl.BlockSpec((1, 1, _D), lambda l, t: (l, 0, 0)),
            pl.BlockSpec((1, 1, _D), lambda l, t: (l, 0, 0)),
            pl.BlockSpec((_D, _D // 2), lambda l, t: (0, 0)),
            pl.BlockSpec((1, _D // 2), lambda l, t: (0, 0)),
            pl.BlockSpec((_D // 2, _D), lambda l, t: (0, 0)),
            pl.BlockSpec((1, _D), lambda l, t: (0, 0)),
        ],
        out_specs=(
            pl.BlockSpec((_N, _D), lambda l, t: (0, 0)),
            pl.BlockSpec((1, _D), lambda l, t: (0, 0)),
        ),
        out_shape=(
            jax.ShapeDtypeStruct((_N, _D), f32),
            jax.ShapeDtypeStruct((1, _D), f32),
        ),
        scratch_shapes=[pltpu.VMEM((_N, _D), f32), pltpu.VMEM((_N, _D), f32)],
    )(A, B, W_ep2_bf, nodes, Wc, bc.reshape(_L, 1, _D), gd, W_nm,
      b_nm.reshape(_L, 1, _D), ln_g.reshape(_L, 1, _D),
      ln_b.reshape(_L, 1, _D), W_gp1, b_gp1.reshape(1, _D // 2),
      W_gp2, b_gp2.reshape(1, _D))

    edge_index = jnp.stack([jnp.asarray(_SRC), jnp.asarray(_TGT)])

    return x, edge_features, edge_index, graph_embedding.reshape(_D)


# speedup vs baseline: 10.4113x; 1.1098x over previous
import numpy as np
import jax
import jax.numpy as jnp
from jax.experimental import pallas as pl
from jax.experimental.pallas import tpu as pltpu

_N = 128
_D = 1024
_ED = 256
_L = 3
_E = _N * (_N - 1)
_TI = 16
_TE = _TI * (_N - 1)
_NE = _E // _TE
_TL = 16
_NT = _N // _TL


def _gelu(x):
    return x * (0.5 * jax.lax.erf(x * np.float32(1.0 / np.sqrt(2.0))) + 0.5)


def _bf(x):
    return x.astype(jnp.bfloat16)


def _dot(a, b):
    return jnp.dot(a, b, preferred_element_type=jnp.float32)


def _prep_body(tracks_ref, W_ne_ref, b_ne_ref, W_ep1_ref, b_ep1_ref,
               W_ep2_ref, b_ep2_ref, W_ee_ref, b_ee_ref, W_em_ref, b_em_ref,
               nodes_ref, a_ref, b_ref, wc_ref, bc_ref, gd_ref, bee_ref):
    nodes = _dot(tracks_ref[...], W_ne_ref[...]) + b_ne_ref[...]
    nodes_ref[...] = nodes
    a = _dot(nodes, W_ep1_ref[:_D, :]) + b_ep1_ref[...]
    b = _dot(nodes, W_ep1_ref[_D:, :])
    a_ref[...] = a
    b_ref[...] = b
    bee = _dot(b_ep2_ref[...], W_ee_ref[...]) + b_ee_ref[...]
    bee_ref[...] = bee
    efd = _dot(_bf(_gelu(a + b)), _bf(W_ep2_ref[...]))
    for l in range(_L):
        wc = _dot(_bf(W_ee_ref[...]), _bf(W_em_ref[l]))
        bc = _dot(bee, W_em_ref[l]) + b_em_ref[pl.ds(l, 1), :]
        wc_ref[l] = _bf(wc)
        bc_ref[pl.ds(l, 1), :] = bc
        gd_ref[l] = _gelu(_dot(_bf(efd), _bf(wc)) + bc)


def _edges_body(iidx_ref, jidx_ref, a_ref, b_ref,
                W_ep2_ref, W_ee_ref, bee_ref, out_ref):
    col = jax.lax.broadcasted_iota(jnp.int32, (_TE, _N), 1)
    pi = (col == iidx_ref[0, 0, :][:, None]).astype(jnp.bfloat16)
    pj = (col == jidx_ref[0, 0, :][:, None]).astype(jnp.bfloat16)
    pre = _dot(pi, a_ref[...]) + _dot(pj, b_ref[...])
    ef0 = _dot(_bf(_gelu(pre)), W_ep2_ref[...])
    out_ref[...] = _dot(_bf(ef0), W_ee_ref[...]) + bee_ref[...]


def _layers_body(a_ref, b_ref, W_ep2_ref, nodes_ref,
                 wc_ref, bc_ref, gd_ref, wnm_ref, b_nm_ref, ln_g_ref,
                 ln_b_ref, W_gp1_ref, b_gp1_ref, W_gp2_ref, b_gp2_ref,
                 x_out_ref, ge_out_ref, x_ref, acc_ref):
    l = pl.program_id(0)
    t = pl.program_id(1)

    @pl.when((l == 0) & (t == 0))
    def _():
        x_ref[...] = nodes_ref[...]

    a = a_ref[...]
    b = b_ref[...]
    pre = a[:, None, :] + b[None, :, :]
    ef0 = _dot(_bf(_gelu(pre.reshape(_TL * _N, _ED))), W_ep2_ref[...])
    g = _gelu(_dot(_bf(ef0), wc_ref[0]) + bc_ref[0])
    g = g.reshape(_TL, _N, _D)
    xt = x_ref[pl.ds(t * _TL, _TL), :]
    contrib = jnp.sum(g * xt[:, None, :], axis=0)

    @pl.when(t == 0)
    def _():
        acc_ref[...] = contrib - x_ref[...] * gd_ref[0]

    @pl.when(t > 0)
    def _():
        acc_ref[...] = acc_ref[...] + contrib

    @pl.when(t == _NT - 1)
    def _():
        agg = acc_ref[...] * (1.0 / 127.0)
        x = x_ref[...]
        wnm = wnm_ref[0]
        h = (_dot(_bf(x), _bf(wnm[:_D, :]))
             + _dot(_bf(agg), _bf(wnm[_D:, :]))
             + b_nm_ref[0])
        mu = jnp.mean(h, axis=1, keepdims=True)
        var = jnp.mean((h - mu) * (h - mu), axis=1, keepdims=True)
        hn = (h - mu) / jnp.sqrt(var + 1e-5) * ln_g_ref[0] + ln_b_ref[0]
        xn = _gelu(hn)
        x_ref[...] = xn

        @pl.when(l == _L - 1)
        def _():
            x_out_ref[...] = xn
            gmean = jnp.mean(xn, axis=0, keepdims=True)
            hp = _gelu(_dot(gmean, W_gp1_ref[...]) + b_gp1_ref[...])
            ge_out_ref[...] = _dot(hp, W_gp2_ref[...]) + b_gp2_ref[...]


_ii = np.repeat(np.arange(_N), _N)
_jj = np.tile(np.arange(_N), _N)
_msk = _ii != _jj
_SRC = np.ascontiguousarray(_ii[_msk]).astype(np.int32)
_TGT = np.ascontiguousarray(_jj[_msk]).astype(np.int32)
_IIDX = _SRC.reshape(_NE, 1, _TE)
_JIDX = _TGT.reshape(_NE, 1, _TE)


def kernel(tracks, W_ne, b_ne, W_ep1, b_ep1, W_ep2, b_ep2, W_ee, b_ee,
           W_nm, b_nm, ln_g, ln_b, W_em, b_em, W_gp1, b_gp1, W_gp2, b_gp2):
    f32 = jnp.float32
    bf16 = jnp.bfloat16

    nodes, A, B, Wc, bc, gd, bee = pl.pallas_call(
        _prep_body,
        out_shape=(
            jax.ShapeDtypeStruct((_N, _D), f32),
            jax.ShapeDtypeStruct((_N, _ED), f32),
            jax.ShapeDtypeStruct((_N, _ED), f32),
            jax.ShapeDtypeStruct((_L, _ED, _D), bf16),
            jax.ShapeDtypeStruct((_L, _D), f32),
            jax.ShapeDtypeStruct((_L, _N, _D), f32),
            jax.ShapeDtypeStruct((1, _D), f32),
        ),
    )(tracks, W_ne, b_ne.reshape(1, _D), W_ep1, b_ep1.reshape(1, _ED),
      W_ep2, b_ep2.reshape(1, _ED), W_ee, b_ee.reshape(1, _D), W_em, b_em)

    W_ep2_bf = W_ep2.astype(bf16)

    edge_features = pl.pallas_call(
        _edges_body,
        grid=(_NE,),
        in_specs=[
            pl.BlockSpec((1, 1, _TE), lambda t: (t, 0, 0)),
            pl.BlockSpec((1, 1, _TE), lambda t: (t, 0, 0)),
            pl.BlockSpec((_N, _ED), lambda t: (0, 0)),
            pl.BlockSpec((_N, _ED), lambda t: (0, 0)),
            pl.BlockSpec((_ED, _ED), lambda t: (0, 0)),
            pl.BlockSpec((_ED, _D), lambda t: (0, 0)),
            pl.BlockSpec((1, _D), lambda t: (0, 0)),
        ],
        out_specs=pl.BlockSpec((_TE, _D), lambda t: (t, 0)),
        out_shape=jax.ShapeDtypeStruct((_E, _D), f32),
    )(jnp.asarray(_IIDX), jnp.asarray(_JIDX), A.astype(bf16), B.astype(bf16),
      W_ep2_bf, W_ee.astype(bf16), bee)

    x, graph_embedding = pl.pallas_call(
        _layers_body,
        grid=(_L, _NT),
        in_specs=[
            pl.BlockSpec((_TL, _ED), lambda l, t: (t, 0)),
            pl.BlockSpec((_N, _ED), lambda l, t: (0, 0)),
            pl.BlockSpec((_ED, _ED), lambda l, t: (0, 0)),
            pl.BlockSpec((_N, _D), lambda l, t: (0, 0)),
            pl.BlockSpec((1, _ED, _D), lambda l, t: (l, 0, 0)),
            pl.BlockSpec((1, 1, _D), lambda l, t: (l, 0, 0)),
            pl.BlockSpec((1, _N, _D), lambda l, t: (l, 0, 0)),
            pl.BlockSpec((1, 2 * _D, _D), lambda l, t: (l, 0, 0)),
            pl.BlockSpec((1, 1, _D), lambda l, t: (l, 0, 0)),
            pl.BlockSpec((1, 1, _D), lambda l, t: (l, 0, 0)),
            pl.BlockSpec((1, 1, _D), lambda l, t: (l, 0, 0)),
            pl.BlockSpec((_D, _D // 2), lambda l, t: (0, 0)),
            pl.BlockSpec((1, _D // 2), lambda l, t: (0, 0)),
            pl.BlockSpec((_D // 2, _D), lambda l, t: (0, 0)),
            pl.BlockSpec((1, _D), lambda l, t: (0, 0)),
        ],
        out_specs=(
            pl.BlockSpec((_N, _D), lambda l, t: (0, 0)),
            pl.BlockSpec((1, _D), lambda l, t: (0, 0)),
        ),
        out_shape=(
            jax.ShapeDtypeStruct((_N, _D), f32),
            jax.ShapeDtypeStruct((1, _D), f32),
        ),
        scratch_shapes=[pltpu.VMEM((_N, _D), f32), pltpu.VMEM((_N, _D), f32)],
    )(A, B, W_ep2_bf, nodes, Wc, bc.reshape(_L, 1, _D), gd, W_nm,
      b_nm.reshape(_L, 1, _D), ln_g.reshape(_L, 1, _D),
      ln_b.reshape(_L, 1, _D), W_gp1, b_gp1.reshape(1, _D // 2),
      W_gp2, b_gp2.reshape(1, _D))

    edge_index = jnp.stack([jnp.asarray(_SRC), jnp.asarray(_TGT)])

    return x, edge_features, edge_index, graph_embedding.reshape(_D)


# speedup vs baseline: 10.7600x; 1.0335x over previous
import numpy as np
import jax
import jax.numpy as jnp
from jax.experimental import pallas as pl
from jax.experimental.pallas import tpu as pltpu

_N = 128
_D = 1024
_ED = 256
_L = 3
_E = _N * (_N - 1)
_TI = 32
_TE = _TI * (_N - 1)
_NE = _E // _TE
_TL = 32
_NT = _N // _TL


def _gelu(x):
    return x * (0.5 * jax.lax.erf(x * np.float32(1.0 / np.sqrt(2.0))) + 0.5)


def _bf(x):
    return x.astype(jnp.bfloat16)


def _dot(a, b):
    return jnp.dot(a, b, preferred_element_type=jnp.float32)


def _prep_body(tracks_ref, W_ne_ref, b_ne_ref, W_ep1_ref, b_ep1_ref,
               W_ep2_ref, b_ep2_ref, W_ee_ref, b_ee_ref, W_em_ref, b_em_ref,
               nodes_ref, a_ref, b_ref, wc_ref, bc_ref, gd_ref, bee_ref):
    nodes = _dot(tracks_ref[...], W_ne_ref[...]) + b_ne_ref[...]
    nodes_ref[...] = nodes
    a = _dot(nodes, W_ep1_ref[:_D, :]) + b_ep1_ref[...]
    b = _dot(nodes, W_ep1_ref[_D:, :])
    a_ref[...] = a
    b_ref[...] = b
    bee = _dot(b_ep2_ref[...], W_ee_ref[...]) + b_ee_ref[...]
    bee_ref[...] = bee
    efd = _dot(_bf(_gelu(a + b)), _bf(W_ep2_ref[...]))
    for l in range(_L):
        wc = _dot(_bf(W_ee_ref[...]), _bf(W_em_ref[l]))
        bc = _dot(bee, W_em_ref[l]) + b_em_ref[pl.ds(l, 1), :]
        wc_ref[l] = _bf(wc)
        bc_ref[pl.ds(l, 1), :] = bc
        gd_ref[l] = _gelu(_dot(_bf(efd), _bf(wc)) + bc)


def _edges_body(iidx_ref, jidx_ref, a_ref, b_ref,
                W_ep2_ref, W_ee_ref, bee_ref, out_ref):
    col = jax.lax.broadcasted_iota(jnp.int32, (_TE, _N), 1)
    pi = (col == iidx_ref[0, 0, :][:, None]).astype(jnp.bfloat16)
    pj = (col == jidx_ref[0, 0, :][:, None]).astype(jnp.bfloat16)
    pre = _dot(pi, a_ref[...]) + _dot(pj, b_ref[...])
    ef0 = _dot(_bf(_gelu(pre)), W_ep2_ref[...])
    out_ref[...] = _dot(_bf(ef0), W_ee_ref[...]) + bee_ref[...]


def _layers_body(a_ref, b_ref, W_ep2_ref, nodes_ref,
                 wc_ref, bc_ref, gd_ref, wnm_ref, b_nm_ref, ln_g_ref,
                 ln_b_ref, W_gp1_ref, b_gp1_ref, W_gp2_ref, b_gp2_ref,
                 x_out_ref, ge_out_ref, x_ref, acc_ref):
    l = pl.program_id(0)
    t = pl.program_id(1)

    @pl.when((l == 0) & (t == 0))
    def _():
        x_ref[...] = nodes_ref[...]

    a = a_ref[...]
    b = b_ref[...]
    pre = a[:, None, :] + b[None, :, :]
    ef0 = _dot(_bf(_gelu(pre.reshape(_TL * _N, _ED))), W_ep2_ref[...])
    g = _gelu(_dot(_bf(ef0), wc_ref[0]) + bc_ref[0])
    g = g.reshape(_TL, _N, _D)
    xt = x_ref[pl.ds(t * _TL, _TL), :]
    contrib = jnp.sum(g * xt[:, None, :], axis=0)

    @pl.when(t == 0)
    def _():
        acc_ref[...] = contrib - x_ref[...] * gd_ref[0]

    @pl.when(t > 0)
    def _():
        acc_ref[...] = acc_ref[...] + contrib

    @pl.when(t == _NT - 1)
    def _():
        agg = acc_ref[...] * (1.0 / 127.0)
        x = x_ref[...]
        wnm = wnm_ref[0]
        h = (_dot(_bf(x), _bf(wnm[:_D, :]))
             + _dot(_bf(agg), _bf(wnm[_D:, :]))
             + b_nm_ref[0])
        mu = jnp.mean(h, axis=1, keepdims=True)
        var = jnp.mean((h - mu) * (h - mu), axis=1, keepdims=True)
        hn = (h - mu) / jnp.sqrt(var + 1e-5) * ln_g_ref[0] + ln_b_ref[0]
        xn = _gelu(hn)
        x_ref[...] = xn

        @pl.when(l == _L - 1)
        def _():
            x_out_ref[...] = xn
            gmean = jnp.mean(xn, axis=0, keepdims=True)
            hp = _gelu(_dot(gmean, W_gp1_ref[...]) + b_gp1_ref[...])
            ge_out_ref[...] = _dot(hp, W_gp2_ref[...]) + b_gp2_ref[...]


_ii = np.repeat(np.arange(_N), _N)
_jj = np.tile(np.arange(_N), _N)
_msk = _ii != _jj
_SRC = np.ascontiguousarray(_ii[_msk]).astype(np.int32)
_TGT = np.ascontiguousarray(_jj[_msk]).astype(np.int32)
_IIDX = _SRC.reshape(_NE, 1, _TE)
_JIDX = _TGT.reshape(_NE, 1, _TE)


def kernel(tracks, W_ne, b_ne, W_ep1, b_ep1, W_ep2, b_ep2, W_ee, b_ee,
           W_nm, b_nm, ln_g, ln_b, W_em, b_em, W_gp1, b_gp1, W_gp2, b_gp2):
    f32 = jnp.float32
    bf16 = jnp.bfloat16

    nodes, A, B, Wc, bc, gd, bee = pl.pallas_call(
        _prep_body,
        out_shape=(
            jax.ShapeDtypeStruct((_N, _D), f32),
            jax.ShapeDtypeStruct((_N, _ED), f32),
            jax.ShapeDtypeStruct((_N, _ED), f32),
            jax.ShapeDtypeStruct((_L, _ED, _D), bf16),
            jax.ShapeDtypeStruct((_L, _D), f32),
            jax.ShapeDtypeStruct((_L, _N, _D), f32),
            jax.ShapeDtypeStruct((1, _D), f32),
        ),
    )(tracks, W_ne, b_ne.reshape(1, _D), W_ep1, b_ep1.reshape(1, _ED),
      W_ep2, b_ep2.reshape(1, _ED), W_ee, b_ee.reshape(1, _D), W_em, b_em)

    W_ep2_bf = W_ep2.astype(bf16)

    edge_features = pl.pallas_call(
        _edges_body,
        grid=(_NE,),
        in_specs=[
            pl.BlockSpec((1, 1, _TE), lambda t: (t, 0, 0)),
            pl.BlockSpec((1, 1, _TE), lambda t: (t, 0, 0)),
            pl.BlockSpec((_N, _ED), lambda t: (0, 0)),
            pl.BlockSpec((_N, _ED), lambda t: (0, 0)),
            pl.BlockSpec((_ED, _ED), lambda t: (0, 0)),
            pl.BlockSpec((_ED, _D), lambda t: (0, 0)),
            pl.BlockSpec((1, _D), lambda t: (0, 0)),
        ],
        out_specs=pl.BlockSpec((_TE, _D), lambda t: (t, 0)),
        out_shape=jax.ShapeDtypeStruct((_E, _D), f32),
    )(jnp.asarray(_IIDX), jnp.asarray(_JIDX), A.astype(bf16), B.astype(bf16),
      W_ep2_bf, W_ee.astype(bf16), bee)

    x, graph_embedding = pl.pallas_call(
        _layers_body,
        grid=(_L, _NT),
        in_specs=[
            pl.BlockSpec((_TL, _ED), lambda l, t: (t, 0)),
            pl.BlockSpec((_N, _ED), lambda l, t: (0, 0)),
            pl.BlockSpec((_ED, _ED), lambda l, t: (0, 0)),
            pl.BlockSpec((_N, _D), lambda l, t: (0, 0)),
            pl.BlockSpec((1, _ED, _D), lambda l, t: (l, 0, 0)),
            pl.BlockSpec((1, 1, _D), lambda l, t: (l, 0, 0)),
            pl.BlockSpec((1, _N, _D), lambda l, t: (l, 0, 0)),
            pl.BlockSpec((1, 2 * _D, _D), lambda l, t: (l, 0, 0)),
            pl.BlockSpec((1, 1, _D), lambda l, t: (l, 0, 0)),
            pl.BlockSpec((1, 1, _D), lambda l, t: (l, 0, 0)),
            pl.BlockSpec((1, 1, _D), lambda l, t: (l, 0, 0)),
            pl.BlockSpec((_D, _D // 2), lambda l, t: (0, 0)),
            pl.BlockSpec((1, _D // 2), lambda l, t: (0, 0)),
            pl.BlockSpec((_D // 2, _D), lambda l, t: (0, 0)),
            pl.BlockSpec((1, _D), lambda l, t: (0, 0)),
        ],
        out_specs=(
            pl.BlockSpec((_N, _D), lambda l, t: (0, 0)),
            pl.BlockSpec((1, _D), lambda l, t: (0, 0)),
        ),
        out_shape=(
            jax.ShapeDtypeStruct((_N, _D), f32),
            jax.ShapeDtypeStruct((1, _D), f32),
        ),
        scratch_shapes=[pltpu.VMEM((_N, _D), f32), pltpu.VMEM((_N, _D), f32)],
    )(A, B, W_ep2_bf, nodes, Wc, bc.reshape(_L, 1, _D), gd, W_nm,
      b_nm.reshape(_L, 1, _D), ln_g.reshape(_L, 1, _D),
      ln_b.reshape(_L, 1, _D), W_gp1, b_gp1.reshape(1, _D // 2),
      W_gp2, b_gp2.reshape(1, _D))

    edge_index = jnp.stack([jnp.asarray(_SRC), jnp.asarray(_TGT)])

    return x, edge_features, edge_index, graph_embedding.reshape(_D)


# speedup vs baseline: 11.2199x; 1.0427x over previous
import numpy as np
import jax
import jax.numpy as jnp
from jax.experimental import pallas as pl
from jax.experimental.pallas import tpu as pltpu

_N = 128
_D = 1024
_ED = 256
_L = 3
_E = _N * (_N - 1)
_TI = 32
_TE = _TI * (_N - 1)
_NE = _E // _TE
_TL = 32
_NT = _N // _TL


def _gelu(x):
    return x * (0.5 * jax.lax.erf(x * np.float32(1.0 / np.sqrt(2.0))) + 0.5)


def _bf(x):
    return x.astype(jnp.bfloat16)


def _dot(a, b):
    return jnp.dot(a, b, preferred_element_type=jnp.float32)


def _prep_body(tracks_ref, W_ne_ref, b_ne_ref, W_ep1_ref, b_ep1_ref,
               W_ep2_ref, b_ep2_ref, W_ee_ref, b_ee_ref, W_em_ref, b_em_ref,
               nodes_ref, a_ref, b_ref, wc_ref, bc_ref, gd_ref, bee_ref,
               abf_ref, bbf_ref, wp2b_ref, weeb_ref):
    nodes = _dot(tracks_ref[...], W_ne_ref[...]) + b_ne_ref[...]
    nodes_ref[...] = nodes
    a = _dot(nodes, W_ep1_ref[:_D, :]) + b_ep1_ref[...]
    b = _dot(nodes, W_ep1_ref[_D:, :])
    a_ref[...] = a
    b_ref[...] = b
    bee = _dot(b_ep2_ref[...], W_ee_ref[...]) + b_ee_ref[...]
    bee_ref[...] = bee
    abf_ref[...] = _bf(a)
    bbf_ref[...] = _bf(b)
    wp2b_ref[...] = _bf(W_ep2_ref[...])
    weeb_ref[...] = _bf(W_ee_ref[...])
    efd = _dot(_bf(_gelu(a + b)), _bf(W_ep2_ref[...]))
    for l in range(_L):
        wc = _dot(_bf(W_ee_ref[...]), _bf(W_em_ref[l]))
        bc = _dot(bee, W_em_ref[l]) + b_em_ref[pl.ds(l, 1), :]
        wc_ref[l] = _bf(wc)
        bc_ref[pl.ds(l, 1), :] = bc
        gd_ref[l] = _gelu(_dot(_bf(efd), _bf(wc)) + bc)


def _edges_body(iidx_ref, jidx_ref, a_ref, b_ref,
                W_ep2_ref, W_ee_ref, bee_ref, out_ref):
    col = jax.lax.broadcasted_iota(jnp.int32, (_TE, _N), 1)
    pi = (col == iidx_ref[0, 0, :][:, None]).astype(jnp.bfloat16)
    pj = (col == jidx_ref[0, 0, :][:, None]).astype(jnp.bfloat16)
    pre = _dot(pi, a_ref[...]) + _dot(pj, b_ref[...])
    ef0 = _dot(_bf(_gelu(pre)), W_ep2_ref[...])
    out_ref[...] = _dot(_bf(ef0), W_ee_ref[...]) + bee_ref[...]


def _layers_body(a_ref, b_ref, W_ep2_ref, nodes_ref,
                 wc_ref, bc_ref, gd_ref, wnm_ref, b_nm_ref, ln_g_ref,
                 ln_b_ref, W_gp1_ref, b_gp1_ref, W_gp2_ref, b_gp2_ref,
                 x_out_ref, ge_out_ref, x_ref, acc_ref):
    l = pl.program_id(0)
    t = pl.program_id(1)

    @pl.when((l == 0) & (t == 0))
    def _():
        x_ref[...] = nodes_ref[...]

    a = a_ref[...]
    b = b_ref[...]
    xt = x_ref[pl.ds(t * _TL, _TL), :]
    _C = 8
    contrib = None
    for c in range(_TL // _C):
        ac = a[c * _C:(c + 1) * _C, :]
        pre = ac[:, None, :] + b[None, :, :]
        ef0 = _dot(_bf(_gelu(pre.reshape(_C * _N, _ED))), W_ep2_ref[...])
        g = _gelu(_dot(_bf(ef0), wc_ref[0]) + bc_ref[0])
        g = g.reshape(_C, _N, _D)
        xc = xt[c * _C:(c + 1) * _C, :]
        part = jnp.sum(g * xc[:, None, :], axis=0)
        contrib = part if contrib is None else contrib + part

    @pl.when(t == 0)
    def _():
        acc_ref[...] = contrib - x_ref[...] * gd_ref[0]

    @pl.when(t > 0)
    def _():
        acc_ref[...] = acc_ref[...] + contrib

    @pl.when(t == _NT - 1)
    def _():
        agg = acc_ref[...] * (1.0 / 127.0)
        x = x_ref[...]
        wnm = wnm_ref[0]
        h = (_dot(_bf(x), _bf(wnm[:_D, :]))
             + _dot(_bf(agg), _bf(wnm[_D:, :]))
             + b_nm_ref[0])
        mu = jnp.mean(h, axis=1, keepdims=True)
        var = jnp.mean((h - mu) * (h - mu), axis=1, keepdims=True)
        hn = (h - mu) / jnp.sqrt(var + 1e-5) * ln_g_ref[0] + ln_b_ref[0]
        xn = _gelu(hn)
        x_ref[...] = xn

        @pl.when(l == _L - 1)
        def _():
            x_out_ref[...] = xn
            gmean = jnp.mean(xn, axis=0, keepdims=True)
            hp = _gelu(_dot(gmean, W_gp1_ref[...]) + b_gp1_ref[...])
            ge_out_ref[...] = _dot(hp, W_gp2_ref[...]) + b_gp2_ref[...]


_ii = np.repeat(np.arange(_N), _N)
_jj = np.tile(np.arange(_N), _N)
_msk = _ii != _jj
_SRC = np.ascontiguousarray(_ii[_msk]).astype(np.int32)
_TGT = np.ascontiguousarray(_jj[_msk]).astype(np.int32)
_IIDX = _SRC.reshape(_NE, 1, _TE)
_JIDX = _TGT.reshape(_NE, 1, _TE)


def kernel(tracks, W_ne, b_ne, W_ep1, b_ep1, W_ep2, b_ep2, W_ee, b_ee,
           W_nm, b_nm, ln_g, ln_b, W_em, b_em, W_gp1, b_gp1, W_gp2, b_gp2):
    f32 = jnp.float32
    bf16 = jnp.bfloat16

    (nodes, A, B, Wc, bc, gd, bee, A_bf, B_bf, W_ep2_bf,
     W_ee_bf) = pl.pallas_call(
        _prep_body,
        out_shape=(
            jax.ShapeDtypeStruct((_N, _D), f32),
            jax.ShapeDtypeStruct((_N, _ED), f32),
            jax.ShapeDtypeStruct((_N, _ED), f32),
            jax.ShapeDtypeStruct((_L, _ED, _D), bf16),
            jax.ShapeDtypeStruct((_L, _D), f32),
            jax.ShapeDtypeStruct((_L, _N, _D), f32),
            jax.ShapeDtypeStruct((1, _D), f32),
            jax.ShapeDtypeStruct((_N, _ED), bf16),
            jax.ShapeDtypeStruct((_N, _ED), bf16),
            jax.ShapeDtypeStruct((_ED, _ED), bf16),
            jax.ShapeDtypeStruct((_ED, _D), bf16),
        ),
    )(tracks, W_ne, b_ne.reshape(1, _D), W_ep1, b_ep1.reshape(1, _ED),
      W_ep2, b_ep2.reshape(1, _ED), W_ee, b_ee.reshape(1, _D), W_em, b_em)

    edge_features = pl.pallas_call(
        _edges_body,
        grid=(_NE,),
        in_specs=[
            pl.BlockSpec((1, 1, _TE), lambda t: (t, 0, 0)),
            pl.BlockSpec((1, 1, _TE), lambda t: (t, 0, 0)),
            pl.BlockSpec((_N, _ED), lambda t: (0, 0)),
            pl.BlockSpec((_N, _ED), lambda t: (0, 0)),
            pl.BlockSpec((_ED, _ED), lambda t: (0, 0)),
            pl.BlockSpec((_ED, _D), lambda t: (0, 0)),
            pl.BlockSpec((1, _D), lambda t: (0, 0)),
        ],
        out_specs=pl.BlockSpec((_TE, _D), lambda t: (t, 0)),
        out_shape=jax.ShapeDtypeStruct((_E, _D), f32),
    )(jnp.asarray(_IIDX), jnp.asarray(_JIDX), A_bf, B_bf,
      W_ep2_bf, W_ee_bf, bee)

    x, graph_embedding = pl.pallas_call(
        _layers_body,
        grid=(_L, _NT),
        in_specs=[
            pl.BlockSpec((_TL, _ED), lambda l, t: (t, 0)),
            pl.BlockSpec((_N, _ED), lambda l, t: (0, 0)),
            pl.BlockSpec((_ED, _ED), lambda l, t: (0, 0)),
            pl.BlockSpec((_N, _D), lambda l, t: (0, 0)),
            pl.BlockSpec((1, _ED, _D), lambda l, t: (l, 0, 0)),
            pl.BlockSpec((1, 1, _D), lambda l, t: (l, 0, 0)),
            pl.BlockSpec((1, _N, _D), lambda l, t: (l, 0, 0)),
            pl.BlockSpec((1, 2 * _D, _D), lambda l, t: (l, 0, 0)),
            pl.BlockSpec((1, 1, _D), lambda l, t: (l, 0, 0)),
            pl.BlockSpec((1, 1, _D), lambda l, t: (l, 0, 0)),
            pl.BlockSpec((1, 1, _D), lambda l, t: (l, 0, 0)),
            pl.BlockSpec((_D, _D // 2), lambda l, t: (0, 0)),
            pl.BlockSpec((1, _D // 2), lambda l, t: (0, 0)),
            pl.BlockSpec((_D // 2, _D), lambda l, t: (0, 0)),
            pl.BlockSpec((1, _D), lambda l, t: (0, 0)),
        ],
        out_specs=(
            pl.BlockSpec((_N, _D), lambda l, t: (0, 0)),
            pl.BlockSpec((1, _D), lambda l, t: (0, 0)),
        ),
        out_shape=(
            jax.ShapeDtypeStruct((_N, _D), f32),
            jax.ShapeDtypeStruct((1, _D), f32),
        ),
        scratch_shapes=[pltpu.VMEM((_N, _D), f32), pltpu.VMEM((_N, _D), f32)],
    )(A, B, W_ep2_bf, nodes, Wc, bc.reshape(_L, 1, _D), gd, W_nm,
      b_nm.reshape(_L, 1, _D), ln_g.reshape(_L, 1, _D),
      ln_b.reshape(_L, 1, _D), W_gp1, b_gp1.reshape(1, _D // 2),
      W_gp2, b_gp2.reshape(1, _D))

    edge_index = jnp.stack([jnp.asarray(_SRC), jnp.asarray(_TGT)])

    return x, edge_features, edge_index, graph_embedding.reshape(_D)


# speedup vs baseline: 11.6177x; 1.0355x over previous
import numpy as np
import jax
import jax.numpy as jnp
from jax.experimental import pallas as pl
from jax.experimental.pallas import tpu as pltpu

_N = 128
_D = 1024
_ED = 256
_L = 3
_E = _N * (_N - 1)
_TL = 32
_NT = _N // _TL
_NET = 2 * _NT
_TE = _E // _NET
_C = 8


def _gelu(x):
    return x * (0.5 * jax.lax.erf(x * np.float32(1.0 / np.sqrt(2.0))) + 0.5)


def _bf(x):
    return x.astype(jnp.bfloat16)


def _dot(a, b):
    return jnp.dot(a, b, preferred_element_type=jnp.float32)


def _prep_body(tracks_ref, W_ne_ref, b_ne_ref, W_ep1_ref, b_ep1_ref,
               W_ep2_ref, b_ep2_ref, W_ee_ref, b_ee_ref, W_em_ref, b_em_ref,
               nodes_ref, a_ref, b_ref, wc_ref, bc_ref, gd_ref, bee_ref,
               abf_ref, bbf_ref, wp2b_ref, weeb_ref, efd_ref, bee_s):
    l = pl.program_id(0)

    @pl.when(l == 0)
    def _():
        nodes = _dot(tracks_ref[...], W_ne_ref[...]) + b_ne_ref[...]
        nodes_ref[...] = nodes
        a = _dot(nodes, W_ep1_ref[:_D, :]) + b_ep1_ref[...]
        b = _dot(nodes, W_ep1_ref[_D:, :])
        a_ref[...] = a
        b_ref[...] = b
        abf_ref[...] = _bf(a)
        bbf_ref[...] = _bf(b)
        wp2b_ref[...] = _bf(W_ep2_ref[...])
        weeb_ref[...] = _bf(W_ee_ref[...])
        bee = _dot(b_ep2_ref[...], W_ee_ref[...]) + b_ee_ref[...]
        bee_ref[...] = bee
        bee_s[...] = bee
        efd_ref[...] = _bf(_dot(_bf(_gelu(a + b)), _bf(W_ep2_ref[...])))

    wc = _dot(_bf(W_ee_ref[...]), _bf(W_em_ref[0]))
    bc = _dot(bee_s[...], W_em_ref[0]) + b_em_ref[0]
    wc_ref[0] = _bf(wc)
    bc_ref[0] = bc
    gd_ref[0] = _gelu(_dot(efd_ref[...], _bf(wc)) + bc)


def _main_body(iidx_ref, jidx_ref, a_ref, b_ref, abf_ref, bbf_ref,
               wp2b_ref, weeb_ref, bee_ref, nodes_ref,
               wc_ref, bc_ref, gd_ref, wnm_ref, b_nm_ref, ln_g_ref,
               ln_b_ref, W_gp1_ref, b_gp1_ref, W_gp2_ref, b_gp2_ref,
               eout_ref, x_out_ref, ge_out_ref, x_ref, acc_ref):
    l = pl.program_id(0)
    t = pl.program_id(1)

    @pl.when((l == 0) & (t == 0))
    def _():
        x_ref[...] = nodes_ref[...]

    @pl.when(l < 2)
    def _():
        col = jax.lax.broadcasted_iota(jnp.int32, (_TE, _N), 1)
        pi = (col == iidx_ref[0, 0, :][:, None]).astype(jnp.bfloat16)
        pj = (col == jidx_ref[0, 0, :][:, None]).astype(jnp.bfloat16)
        pre = _dot(pi, abf_ref[...]) + _dot(pj, bbf_ref[...])
        ef0 = _dot(_bf(_gelu(pre)), wp2b_ref[...])
        eout_ref[...] = _dot(_bf(ef0), weeb_ref[...]) + bee_ref[...]

    a = a_ref[...]
    b = b_ref[...]
    xt = x_ref[pl.ds(t * _TL, _TL), :]
    contrib = None
    for c in range(_TL // _C):
        ac = a[c * _C:(c + 1) * _C, :]
        pre = ac[:, None, :] + b[None, :, :]
        ef0 = _dot(_bf(_gelu(pre.reshape(_C * _N, _ED))), wp2b_ref[...])
        g = _gelu(_dot(_bf(ef0), wc_ref[0]) + bc_ref[0])
        g = g.reshape(_C, _N, _D)
        xc = xt[c * _C:(c + 1) * _C, :]
        part = jnp.sum(g * xc[:, None, :], axis=0)
        contrib = part if contrib is None else contrib + part

    @pl.when(t == 0)
    def _():
        acc_ref[...] = contrib - x_ref[...] * gd_ref[0]

    @pl.when(t > 0)
    def _():
        acc_ref[...] = acc_ref[...] + contrib

    @pl.when(t == _NT - 1)
    def _():
        agg = acc_ref[...] * (1.0 / 127.0)
        x = x_ref[...]
        wnm = wnm_ref[0]
        h = (_dot(_bf(x), _bf(wnm[:_D, :]))
             + _dot(_bf(agg), _bf(wnm[_D:, :]))
             + b_nm_ref[0])
        mu = jnp.mean(h, axis=1, keepdims=True)
        var = jnp.mean((h - mu) * (h - mu), axis=1, keepdims=True)
        hn = (h - mu) / jnp.sqrt(var + 1e-5) * ln_g_ref[0] + ln_b_ref[0]
        xn = _gelu(hn)
        x_ref[...] = xn

        @pl.when(l == _L - 1)
        def _():
            x_out_ref[...] = xn
            gmean = jnp.mean(xn, axis=0, keepdims=True)
            hp = _gelu(_dot(gmean, W_gp1_ref[...]) + b_gp1_ref[...])
            ge_out_ref[...] = _dot(hp, W_gp2_ref[...]) + b_gp2_ref[...]


_ii = np.repeat(np.arange(_N), _N)
_jj = np.tile(np.arange(_N), _N)
_msk = _ii != _jj
_SRC = np.ascontiguousarray(_ii[_msk]).astype(np.int32)
_TGT = np.ascontiguousarray(_jj[_msk]).astype(np.int32)
_IIDX = _SRC.reshape(_NET, 1, _TE)
_JIDX = _TGT.reshape(_NET, 1, _TE)


def _etile(l, t):
    return jnp.minimum(l * _NT + t, _NET - 1)


def kernel(tracks, W_ne, b_ne, W_ep1, b_ep1, W_ep2, b_ep2, W_ee, b_ee,
           W_nm, b_nm, ln_g, ln_b, W_em, b_em, W_gp1, b_gp1, W_gp2, b_gp2):
    f32 = jnp.float32
    bf16 = jnp.bfloat16

    (nodes, A, B, Wc, bc, gd, bee, A_bf, B_bf, W_ep2_bf,
     W_ee_bf) = pl.pallas_call(
        _prep_body,
        grid=(_L,),
        in_specs=[
            pl.BlockSpec((_N, _D), lambda l: (0, 0)),
            pl.BlockSpec((_D, _D), lambda l: (0, 0)),
            pl.BlockSpec((1, _D), lambda l: (0, 0)),
            pl.BlockSpec((2 * _D, _ED), lambda l: (0, 0)),
            pl.BlockSpec((1, _ED), lambda l: (0, 0)),
            pl.BlockSpec((_ED, _ED), lambda l: (0, 0)),
            pl.BlockSpec((1, _ED), lambda l: (0, 0)),
            pl.BlockSpec((_ED, _D), lambda l: (0, 0)),
            pl.BlockSpec((1, _D), lambda l: (0, 0)),
            pl.BlockSpec((1, _D, _D), lambda l: (l, 0, 0)),
            pl.BlockSpec((1, 1, _D), lambda l: (l, 0, 0)),
        ],
        out_specs=(
            pl.BlockSpec((_N, _D), lambda l: (0, 0)),
            pl.BlockSpec((_N, _ED), lambda l: (0, 0)),
            pl.BlockSpec((_N, _ED), lambda l: (0, 0)),
            pl.BlockSpec((1, _ED, _D), lambda l: (l, 0, 0)),
            pl.BlockSpec((1, 1, _D), lambda l: (l, 0, 0)),
            pl.BlockSpec((1, _N, _D), lambda l: (l, 0, 0)),
            pl.BlockSpec((1, _D), lambda l: (0, 0)),
            pl.BlockSpec((_N, _ED), lambda l: (0, 0)),
            pl.BlockSpec((_N, _ED), lambda l: (0, 0)),
            pl.BlockSpec((_ED, _ED), lambda l: (0, 0)),
            pl.BlockSpec((_ED, _D), lambda l: (0, 0)),
        ),
        out_shape=(
            jax.ShapeDtypeStruct((_N, _D), f32),
            jax.ShapeDtypeStruct((_N, _ED), f32),
            jax.ShapeDtypeStruct((_N, _ED), f32),
            jax.ShapeDtypeStruct((_L, _ED, _D), bf16),
            jax.ShapeDtypeStruct((_L, 1, _D), f32),
            jax.ShapeDtypeStruct((_L, _N, _D), f32),
            jax.ShapeDtypeStruct((1, _D), f32),
            jax.ShapeDtypeStruct((_N, _ED), bf16),
            jax.ShapeDtypeStruct((_N, _ED), bf16),
            jax.ShapeDtypeStruct((_ED, _ED), bf16),
            jax.ShapeDtypeStruct((_ED, _D), bf16),
        ),
        scratch_shapes=[pltpu.VMEM((_N, _ED), bf16), pltpu.VMEM((1, _D), f32)],
    )(tracks, W_ne, b_ne.reshape(1, _D), W_ep1, b_ep1.reshape(1, _ED),
      W_ep2, b_ep2.reshape(1, _ED), W_ee, b_ee.reshape(1, _D), W_em,
      b_em.reshape(_L, 1, _D))

    edge_features, x, graph_embedding = pl.pallas_call(
        _main_body,
        grid=(_L, _NT),
        in_specs=[
            pl.BlockSpec((1, 1, _TE), lambda l, t: (_etile(l, t), 0, 0)),
            pl.BlockSpec((1, 1, _TE), lambda l, t: (_etile(l, t), 0, 0)),
            pl.BlockSpec((_TL, _ED), lambda l, t: (t, 0)),
            pl.BlockSpec((_N, _ED), lambda l, t: (0, 0)),
            pl.BlockSpec((_N, _ED), lambda l, t: (0, 0)),
            pl.BlockSpec((_N, _ED), lambda l, t: (0, 0)),
            pl.BlockSpec((_ED, _ED), lambda l, t: (0, 0)),
            pl.BlockSpec((_ED, _D), lambda l, t: (0, 0)),
            pl.BlockSpec((1, _D), lambda l, t: (0, 0)),
            pl.BlockSpec((_N, _D), lambda l, t: (0, 0)),
            pl.BlockSpec((1, _ED, _D), lambda l, t: (l, 0, 0)),
            pl.BlockSpec((1, 1, _D), lambda l, t: (l, 0, 0)),
            pl.BlockSpec((1, _N, _D), lambda l, t: (l, 0, 0)),
            pl.BlockSpec((1, 2 * _D, _D), lambda l, t: (l, 0, 0)),
            pl.BlockSpec((1, 1, _D), lambda l, t: (l, 0, 0)),
            pl.BlockSpec((1, 1, _D), lambda l, t: (l, 0, 0)),
            pl.BlockSpec((1, 1, _D), lambda l, t: (l, 0, 0)),
            pl.BlockSpec((_D, _D // 2), lambda l, t: (0, 0)),
            pl.BlockSpec((1, _D // 2), lambda l, t: (0, 0)),
            pl.BlockSpec((_D // 2, _D), lambda l, t: (0, 0)),
            pl.BlockSpec((1, _D), lambda l, t: (0, 0)),
        ],
        out_specs=(
            pl.BlockSpec((_TE, _D), lambda l, t: (_etile(l, t), 0)),
            pl.BlockSpec((_N, _D), lambda l, t: (0, 0)),
            pl.BlockSpec((1, _D), lambda l, t: (0, 0)),
        ),
        out_shape=(
            jax.ShapeDtypeStruct((_E, _D), f32),
            jax.ShapeDtypeStruct((_N, _D), f32),
            jax.ShapeDtypeStruct((1, _D), f32),
        ),
        scratch_shapes=[pltpu.VMEM((_N, _D), f32), pltpu.VMEM((_N, _D), f32)],
    )(jnp.asarray(_IIDX), jnp.asarray(_JIDX), A, B, A_bf, B_bf,
      W_ep2_bf, W_ee_bf, bee, nodes, Wc, bc, gd, W_nm,
      b_nm.reshape(_L, 1, _D), ln_g.reshape(_L, 1, _D),
      ln_b.reshape(_L, 1, _D), W_gp1, b_gp1.reshape(1, _D // 2),
      W_gp2, b_gp2.reshape(1, _D))

    edge_index = jnp.stack([jnp.asarray(_SRC), jnp.asarray(_TGT)])

    return x, edge_features, edge_index, graph_embedding.reshape(_D)
